# baseline (speedup 1.0000x reference)
import functools

import jax
import jax.numpy as jnp
from jax import lax
from jax.experimental import pallas as pl
from jax.experimental.pallas import tpu as pltpu
from jax.experimental.pallas import tpu_sc as plsc

BATCH = 16384
EMBED = 64
LANES = 16
NUM_WORKERS = 32
B_PER_W = BATCH // NUM_WORKERS


def _sc_body(uid_hbm, vid_hbm, ut_hbm, vt_hbm, out_hbm,
             uidx_v, vidx_v, urows_v, vrows_v, out_v, sem_u, sem_v):
    wid = lax.axis_index("s") * 2 + lax.axis_index("c")
    base = wid * B_PER_W

    pltpu.sync_copy(uid_hbm.at[pl.ds(base, B_PER_W)], uidx_v)
    pltpu.sync_copy(vid_hbm.at[pl.ds(base, B_PER_W)], vidx_v)
    cu = pltpu.async_copy(ut_hbm.at[uidx_v], urows_v, sem_u)
    cv = pltpu.async_copy(vt_hbm.at[vidx_v], vrows_v, sem_v)
    cu.wait()
    cv.wait()

    row_iota = lax.iota(jnp.int32, LANES)

    def group(g, carry):
        rows = row_iota + g * LANES
        acc = jnp.zeros((LANES,), jnp.float32)
        for d in range(EMBED):
            col = jnp.full((LANES,), d, jnp.int32)
            u = plsc.load_gather(urows_v, [rows, col])
            v = plsc.load_gather(vrows_v, [rows, col])
            acc = acc + u * v
        out_v[pl.ds(g * LANES, LANES)] = acc
        return carry

    lax.fori_loop(0, B_PER_W // LANES, group, 0)
    pltpu.sync_copy(out_v, out_hbm.at[pl.ds(base, B_PER_W)])


@jax.jit
def kernel(user_id, video_id, user_table, video_table):
    mesh = plsc.VectorSubcoreMesh(core_axis_name="c", subcore_axis_name="s")
    run = functools.partial(
        pl.kernel,
        mesh=mesh,
        compiler_params=pltpu.CompilerParams(
            needs_layout_passes=False, use_tc_tiling_on_sc=False),
        out_type=jax.ShapeDtypeStruct((BATCH,), jnp.float32),
        scratch_types=[
            pltpu.VMEM((B_PER_W,), jnp.int32),
            pltpu.VMEM((B_PER_W,), jnp.int32),
            pltpu.VMEM((B_PER_W, EMBED), jnp.float32),
            pltpu.VMEM((B_PER_W, EMBED), jnp.float32),
            pltpu.VMEM((B_PER_W,), jnp.float32),
            pltpu.SemaphoreType.DMA,
            pltpu.SemaphoreType.DMA,
        ],
    )(_sc_body)
    return run(user_id.astype(jnp.int32), video_id.astype(jnp.int32),
               user_table, video_table)


# speedup vs baseline: 1.2055x; 1.2055x over previous
import functools

import jax
import jax.numpy as jnp
from jax import lax
from jax.experimental import pallas as pl
from jax.experimental.pallas import tpu as pltpu
from jax.experimental.pallas import tpu_sc as plsc

BATCH = 16384
EMBED = 64
LANES = 16
NUM_WORKERS = 32
B_PER_W = BATCH // NUM_WORKERS


def _sc_body(uid_hbm, vid_hbm, ut_hbm, vt_hbm, out_hbm,
             uidx_v, vidx_v, urows_v, vrows_v, out_v, sem_u, sem_v):
    wid = lax.axis_index("s") * 2 + lax.axis_index("c")
    base = wid * B_PER_W

    pltpu.sync_copy(uid_hbm.at[pl.ds(base, B_PER_W)], uidx_v)
    pltpu.sync_copy(vid_hbm.at[pl.ds(base, B_PER_W)], vidx_v)
    cu = pltpu.async_copy(ut_hbm.at[uidx_v], urows_v, sem_u)
    cv = pltpu.async_copy(vt_hbm.at[vidx_v], vrows_v, sem_v)
    cu.wait()
    cv.wait()

    row_iota = lax.iota(jnp.int32, LANES)

    def group(g, carry):
        rows = row_iota + g * LANES
        acc = jnp.zeros((LANES,), jnp.float32)
        for d in range(EMBED):
            col = (row_iota + d) & (EMBED - 1)
            u = plsc.load_gather(urows_v, [rows, col])
            v = plsc.load_gather(vrows_v, [rows, col])
            acc = acc + u * v
        out_v[pl.ds(g * LANES, LANES)] = acc
        return carry

    lax.fori_loop(0, B_PER_W // LANES, group, 0)
    pltpu.sync_copy(out_v, out_hbm.at[pl.ds(base, B_PER_W)])


@jax.jit
def kernel(user_id, video_id, user_table, video_table):
    mesh = plsc.VectorSubcoreMesh(core_axis_name="c", subcore_axis_name="s")
    run = functools.partial(
        pl.kernel,
        mesh=mesh,
        compiler_params=pltpu.CompilerParams(
            needs_layout_passes=False, use_tc_tiling_on_sc=False),
        out_type=jax.ShapeDtypeStruct((BATCH,), jnp.float32),
        scratch_types=[
            pltpu.VMEM((B_PER_W,), jnp.int32),
            pltpu.VMEM((B_PER_W,), jnp.int32),
            pltpu.VMEM((B_PER_W, EMBED), jnp.float32),
            pltpu.VMEM((B_PER_W, EMBED), jnp.float32),
            pltpu.VMEM((B_PER_W,), jnp.float32),
            pltpu.SemaphoreType.DMA,
            pltpu.SemaphoreType.DMA,
        ],
    )(_sc_body)
    return run(user_id.astype(jnp.int32), video_id.astype(jnp.int32),
               user_table, video_table)


# speedup vs baseline: 1.2551x; 1.0411x over previous
import functools

import jax
import jax.numpy as jnp
from jax import lax
from jax.experimental import pallas as pl
from jax.experimental.pallas import tpu as pltpu
from jax.experimental.pallas import tpu_sc as plsc

BATCH = 16384
EMBED = 64
ROW = 2 * EMBED
LANES = 16
NUM_WORKERS = 32
B_PER_W = BATCH // NUM_WORKERS
NPASS = 2
B_PER_P = B_PER_W // NPASS


def _sc_body(uid_hbm, vid_hbm, ut_hbm, vt_hbm, out_hbm,
             uidx_v, vidx_v, urows_v, vrows_v, out_v, sem_u, sem_v):
    wid = lax.axis_index("s") * 2 + lax.axis_index("c")
    base = wid * B_PER_W

    pltpu.sync_copy(uid_hbm.at[pl.ds(base, B_PER_W)], uidx_v)
    pltpu.sync_copy(vid_hbm.at[pl.ds(base, B_PER_W)], vidx_v)

    row_iota = lax.iota(jnp.int32, LANES)

    for p in range(NPASS):
        cu = pltpu.async_copy(
            ut_hbm.at[uidx_v.at[pl.ds(p * B_PER_P, B_PER_P)]], urows_v, sem_u)
        cv = pltpu.async_copy(
            vt_hbm.at[vidx_v.at[pl.ds(p * B_PER_P, B_PER_P)]], vrows_v, sem_v)
        cu.wait()
        cv.wait()

        def group(g, carry):
            rows = row_iota + g * LANES
            acc = jnp.zeros((LANES,), jnp.float32)
            for d in range(EMBED):
                col = (row_iota + d) & (EMBED - 1)
                u = plsc.load_gather(urows_v, [rows, col])
                v = plsc.load_gather(vrows_v, [rows, col])
                acc = acc + u * v
            out_v[pl.ds(p * B_PER_P + g * LANES, LANES)] = acc
            return carry

        lax.fori_loop(0, B_PER_P // LANES, group, 0)

    pltpu.sync_copy(out_v, out_hbm.at[pl.ds(base, B_PER_W)])


@jax.jit
def kernel(user_id, video_id, user_table, video_table):
    uid = user_id.astype(jnp.int32)
    vid = video_id.astype(jnp.int32)
    ut128 = jnp.pad(user_table, ((0, 0), (0, ROW - EMBED)))
    vt128 = jnp.pad(video_table, ((0, 0), (0, ROW - EMBED)))
    mesh = plsc.VectorSubcoreMesh(core_axis_name="c", subcore_axis_name="s")
    run = functools.partial(
        pl.kernel,
        mesh=mesh,
        compiler_params=pltpu.CompilerParams(needs_layout_passes=False),
        out_type=jax.ShapeDtypeStruct((BATCH,), jnp.float32),
        scratch_types=[
            pltpu.VMEM((B_PER_W,), jnp.int32),
            pltpu.VMEM((B_PER_W,), jnp.int32),
            pltpu.VMEM((B_PER_P, ROW), jnp.float32),
            pltpu.VMEM((B_PER_P, ROW), jnp.float32),
            pltpu.VMEM((B_PER_W,), jnp.float32),
            pltpu.SemaphoreType.DMA,
            pltpu.SemaphoreType.DMA,
        ],
    )(_sc_body)
    return run(uid, vid, ut128, vt128)
